# baseline (speedup 1.0000x reference)
import jax
import jax.numpy as jnp
from jax import lax
from jax.experimental import pallas as pl
from jax.experimental.pallas import tpu as pltpu
from jax.experimental.pallas import tpu_sc as plsc

B, S, D = 16, 2048, 512
CH = 32
NV = D // 16
HNV = NV // 2
NW = 32
BS_TC = 128
KMAX = B * S // BS_TC
FRAC_P, FRAC_Q = 3, 4


def _scalar_at(vec_ref, i):
    return vec_ref[pl.ds(i, 16)][0]


def _sc_body(x_hbm, n_hbm, m_hbm, part_hbm, nvec_ref, mvec_ref,
             buf0_ref, buf1_ref, part_ref, sem0, sem1):
    c = lax.axis_index("c")
    s = lax.axis_index("s")
    w = s * 2 + c

    pltpu.sync_copy(n_hbm, nvec_ref.at[pl.ds(0, 16)])
    pltpu.sync_copy(m_hbm, mvec_ref.at[pl.ds(0, 16)])

    def tot_body(j, t):
        return t + (_scalar_at(nvec_ref, j) - _scalar_at(mvec_ref, j))
    T = lax.fori_loop(0, B, tot_body, jnp.int32(0))

    lo = w * T // NW
    hi = (w + 1) * T // NW

    zero = jnp.zeros((16,), jnp.float32)

    def zero_body(r, _):
        for j in range(NV):
            part_ref[r, pl.ds(j * 16, 16)] = zero
        return 0
    lax.fori_loop(0, B, zero_body, 0)

    bufs = (buf0_ref, buf1_ref)
    sems = (sem0, sem1)

    def batch_body(i, C):
        n_i = _scalar_at(nvec_ref, i)
        m_i = _scalar_at(mvec_ref, i)
        len_i = n_i - m_i
        a = jnp.maximum(lo, C)
        b = jnp.minimum(hi, C + len_i)

        @pl.when(b > a)
        def _():
            r0 = m_i + (a - C)
            r1 = m_i + (b - C)
            c0 = r0 // CH
            c1 = (r1 + CH - 1) // CH

            def start(chunk, bi):
                @pl.when(chunk < c1)
                def _():
                    pltpu.async_copy(
                        x_hbm.at[i, pl.ds(chunk * CH, CH)], bufs[bi],
                        sems[bi])

            def wait(bi):
                pltpu.make_async_copy(
                    x_hbm.at[i, pl.ds(0, CH)], bufs[bi], sems[bi]).wait()

            start(c0, 0)
            start(c0 + 1, 1)

            def accum_chunk(chunk, bi, accs):
                lo_r = jnp.maximum(r0 - chunk * CH, 0)
                hi_r = jnp.minimum(r1 - chunk * CH, CH)
                buf = bufs[bi]

                @pl.when(chunk < c1)
                def _():
                    wait(bi)

                accs_lo, accs_hi = accs[:HNV], accs[HNV:]

                def row_lo(r, a):
                    return tuple(a[j] + buf[r, pl.ds(j * 16, 16)]
                                 for j in range(HNV))

                def row_hi(r, a):
                    return tuple(a[j] + buf[r, pl.ds((HNV + j) * 16, 16)]
                                 for j in range(HNV))

                accs_lo = lax.fori_loop(lo_r, hi_r, row_lo, accs_lo)
                accs_hi = lax.fori_loop(lo_r, hi_r, row_hi, accs_hi)
                start(chunk + 2, bi)
                return accs_lo + accs_hi

            def pair_body(it, accs):
                chunk = c0 + 2 * it
                accs = accum_chunk(chunk, 0, accs)
                accs = accum_chunk(chunk + 1, 1, accs)
                return accs

            accs0 = tuple(jnp.zeros((16,), jnp.float32) for _ in range(NV))
            npairs = (c1 - c0 + 1) // 2
            accs = lax.fori_loop(0, npairs, pair_body, accs0)
            for j in range(NV):
                part_ref[i, pl.ds(j * 16, 16)] = accs[j]

        return C + len_i

    lax.fori_loop(0, B, batch_body, jnp.int32(0))

    pltpu.sync_copy(part_ref, part_hbm.at[w])


def _tc_dense_body(cb_ref, cr_ref, fl_ref, meta_ref, x_hbm, o_ref,
                   buf0_ref, buf1_ref, sem0, sem1):
    K = meta_ref[0]
    bufs = (buf0_ref, buf1_ref)
    sems = (sem0, sem1)

    def start(g, bi):
        @pl.when(g < K)
        def _():
            row = pl.multiple_of(cr_ref[g], BS_TC)
            pltpu.make_async_copy(
                x_hbm.at[cb_ref[g], pl.ds(row, BS_TC)],
                bufs[bi], sems[bi]).start()

    def wait(bi):
        pltpu.make_async_copy(
            x_hbm.at[0, pl.ds(0, BS_TC)], bufs[bi], sems[bi]).wait()

    start(0, 0)
    start(1, 1)
    o_ref[...] = jnp.zeros_like(o_ref)

    def halfstep(g, bi, acc):
        valid = g < K

        @pl.when(valid)
        def _():
            wait(bi)

        buf = bufs[bi]
        csum = buf[pl.ds(0, 8), :]
        for r in range(8, BS_TC, 8):
            csum = csum + buf[pl.ds(r, 8), :]
        tot = acc + jnp.where(valid, csum, jnp.zeros_like(csum))
        start(g + 2, bi)
        flush = jnp.logical_and(fl_ref[g] == 1, valid)

        @pl.when(flush)
        def _():
            o_ref[pl.ds(cb_ref[g], 1)] = jnp.sum(
                tot, axis=0, keepdims=True).reshape(1, 1, D)

        return jnp.where(flush, jnp.zeros_like(tot), tot)

    def pair_body(it, acc):
        g = 2 * it
        acc = halfstep(g, 0, acc)
        acc = halfstep(g + 1, 1, acc)
        return acc

    npairs = (K + 1) // 2
    lax.fori_loop(0, npairs, pair_body, jnp.zeros((8, D), jnp.float32))


def _combine_body(part_ref, tc_ref, nf_ref, out_ref):
    out_ref[...] = (jnp.sum(part_ref[...], axis=0) + tc_ref[...]) / nf_ref[...]


def kernel(x, N):
    k = (FRAC_P * N) // (FRAC_Q * BS_TC)
    m = k * BS_TC

    K = jnp.sum(k)
    cumk = jnp.cumsum(k)
    cume = cumk - k
    cb = jnp.repeat(jnp.arange(B, dtype=jnp.int32), k,
                    total_repeat_length=KMAX)
    g = jnp.arange(KMAX, dtype=jnp.int32)
    cr = (g - cume[cb]) * BS_TC
    fl = jnp.zeros((KMAX,), jnp.int32).at[
        jnp.where(k > 0, cumk - 1, KMAX + 1)].set(1, mode="drop")
    meta = jnp.reshape(K, (1,)).astype(jnp.int32)

    mesh = plsc.VectorSubcoreMesh(core_axis_name="c", subcore_axis_name="s")
    sc = pl.kernel(
        _sc_body,
        out_type=jax.ShapeDtypeStruct((NW, B, D), jnp.float32),
        mesh=mesh,
        scratch_types=[
            pltpu.VMEM((32,), jnp.int32),
            pltpu.VMEM((32,), jnp.int32),
            pltpu.VMEM((CH, D), jnp.float32),
            pltpu.VMEM((CH, D), jnp.float32),
            pltpu.VMEM((B, D), jnp.float32),
            pltpu.SemaphoreType.DMA,
            pltpu.SemaphoreType.DMA,
        ],
    )
    partials = sc(x, N, m)

    tcsum = pl.pallas_call(
        _tc_dense_body,
        grid_spec=pltpu.PrefetchScalarGridSpec(
            num_scalar_prefetch=4,
            grid=(1,),
            in_specs=[
                pl.BlockSpec(memory_space=pltpu.MemorySpace.HBM),
            ],
            out_specs=pl.BlockSpec(
                (B, 1, D), lambda _, *refs: (0, 0, 0)),
            scratch_shapes=[
                pltpu.VMEM((BS_TC, D), jnp.float32),
                pltpu.VMEM((BS_TC, D), jnp.float32),
                pltpu.SemaphoreType.DMA,
                pltpu.SemaphoreType.DMA,
            ],
        ),
        out_shape=jax.ShapeDtypeStruct((B, 1, D), jnp.float32),
    )(cb, cr, fl, meta, x).reshape(B, D)

    nf = N.astype(jnp.float32).reshape(B, 1)
    return pl.pallas_call(
        _combine_body,
        out_shape=jax.ShapeDtypeStruct((B, D), jnp.float32),
    )(partials, tcsum, nf)


# speedup vs baseline: 1.4339x; 1.4339x over previous
import jax
import jax.numpy as jnp
from jax import lax
from jax.experimental import pallas as pl
from jax.experimental.pallas import tpu as pltpu
from jax.experimental.pallas import tpu_sc as plsc

B, S, D = 16, 2048, 512
CH = 32
SC_NBUF = 4
NV = D // 16
HNV = NV // 2
NW = 32
BS_TC = 128
TC_NBUF = 8
KMAX = B * S // BS_TC
FRAC_P, FRAC_Q = 3, 4


def _scalar_at(vec_ref, i):
    return vec_ref[pl.ds(i, 16)][0]


def _sc_body(x_hbm, n_hbm, m_hbm, part_hbm, nvec_ref, mvec_ref,
             b0, b1, b2, b3, part_ref, s0, s1, s2, s3):
    c = lax.axis_index("c")
    s = lax.axis_index("s")
    w = s * 2 + c

    pltpu.sync_copy(n_hbm, nvec_ref.at[pl.ds(0, 16)])
    pltpu.sync_copy(m_hbm, mvec_ref.at[pl.ds(0, 16)])

    def tot_body(j, t):
        return t + (_scalar_at(nvec_ref, j) - _scalar_at(mvec_ref, j))
    T = lax.fori_loop(0, B, tot_body, jnp.int32(0))

    lo = w * T // NW
    hi = (w + 1) * T // NW

    zero = jnp.zeros((16,), jnp.float32)

    def zero_body(r, _):
        for j in range(NV):
            part_ref[r, pl.ds(j * 16, 16)] = zero
        return 0
    lax.fori_loop(0, B, zero_body, 0)

    bufs = (b0, b1, b2, b3)
    sems = (s0, s1, s2, s3)

    def batch_body(i, C):
        n_i = _scalar_at(nvec_ref, i)
        m_i = _scalar_at(mvec_ref, i)
        len_i = n_i - m_i
        a = jnp.maximum(lo, C)
        b = jnp.minimum(hi, C + len_i)

        @pl.when(b > a)
        def _():
            r0 = m_i + (a - C)
            r1 = m_i + (b - C)
            c0 = r0 // CH
            c1 = (r1 + CH - 1) // CH

            def start(chunk, bi):
                @pl.when(chunk < c1)
                def _():
                    pltpu.async_copy(
                        x_hbm.at[i, pl.ds(chunk * CH, CH)], bufs[bi],
                        sems[bi])

            def wait(bi):
                pltpu.make_async_copy(
                    x_hbm.at[i, pl.ds(0, CH)], bufs[bi], sems[bi]).wait()

            for bi in range(SC_NBUF):
                start(c0 + bi, bi)

            def accum_chunk(chunk, bi, accs):
                lo_r = jnp.maximum(r0 - chunk * CH, 0)
                hi_r = jnp.minimum(r1 - chunk * CH, CH)
                buf = bufs[bi]

                @pl.when(chunk < c1)
                def _():
                    wait(bi)

                accs_lo, accs_hi = accs[:HNV], accs[HNV:]

                def row_lo(r, a):
                    return tuple(a[j] + buf[r, pl.ds(j * 16, 16)]
                                 for j in range(HNV))

                def row_hi(r, a):
                    return tuple(a[j] + buf[r, pl.ds((HNV + j) * 16, 16)]
                                 for j in range(HNV))

                accs_lo = lax.fori_loop(lo_r, hi_r, row_lo, accs_lo)
                accs_hi = lax.fori_loop(lo_r, hi_r, row_hi, accs_hi)
                start(chunk + SC_NBUF, bi)
                return accs_lo + accs_hi

            def ring_body(it, accs):
                chunk = c0 + SC_NBUF * it
                for bi in range(SC_NBUF):
                    accs = accum_chunk(chunk + bi, bi, accs)
                return accs

            accs0 = tuple(jnp.zeros((16,), jnp.float32) for _ in range(NV))
            nrings = (c1 - c0 + SC_NBUF - 1) // SC_NBUF
            accs = lax.fori_loop(0, nrings, ring_body, accs0)
            for j in range(NV):
                part_ref[i, pl.ds(j * 16, 16)] = accs[j]

        return C + len_i

    lax.fori_loop(0, B, batch_body, jnp.int32(0))

    pltpu.sync_copy(part_ref, part_hbm.at[w])


def _tc_dense_body(cb_ref, cr_ref, fl_ref, meta_ref, x_hbm, o_ref, *rest):
    bufs = rest[:TC_NBUF]
    sems = rest[TC_NBUF:]
    K = meta_ref[0]

    def start(g, bi):
        @pl.when(g < K)
        def _():
            row = pl.multiple_of(cr_ref[g], BS_TC)
            pltpu.make_async_copy(
                x_hbm.at[cb_ref[g], pl.ds(row, BS_TC)],
                bufs[bi], sems[bi]).start()

    def wait(bi):
        pltpu.make_async_copy(
            x_hbm.at[0, pl.ds(0, BS_TC)], bufs[bi], sems[bi]).wait()

    for bi in range(TC_NBUF):
        start(bi, bi)
    o_ref[...] = jnp.zeros_like(o_ref)

    def step(g, bi, acc):
        valid = g < K

        @pl.when(valid)
        def _():
            wait(bi)

        buf = bufs[bi]
        ps = [buf[pl.ds(p * 8, 8), :] for p in range(4)]
        for r in range(32, BS_TC, 32):
            for p in range(4):
                ps[p] = ps[p] + buf[pl.ds(r + p * 8, 8), :]
        csum = (ps[0] + ps[1]) + (ps[2] + ps[3])
        tot = acc + jnp.where(valid, csum, jnp.zeros_like(csum))
        start(g + TC_NBUF, bi)
        flush = jnp.logical_and(fl_ref[g] == 1, valid)

        @pl.when(flush)
        def _():
            o_ref[pl.ds(cb_ref[g], 1)] = jnp.sum(
                tot, axis=0, keepdims=True).reshape(1, 1, D)

        return jnp.where(flush, jnp.zeros_like(tot), tot)

    def ring_body(it, acc):
        g = TC_NBUF * it
        for bi in range(TC_NBUF):
            acc = step(g + bi, bi, acc)
        return acc

    nrings = (K + TC_NBUF - 1) // TC_NBUF
    lax.fori_loop(0, nrings, ring_body, jnp.zeros((8, D), jnp.float32))


def _combine_body(part_ref, tc_ref, nf_ref, out_ref):
    out_ref[...] = (jnp.sum(part_ref[...], axis=0) + tc_ref[...]) / nf_ref[...]


def kernel(x, N):
    k = (FRAC_P * N) // (FRAC_Q * BS_TC)
    m = k * BS_TC

    K = jnp.sum(k)
    cumk = jnp.cumsum(k)
    cume = cumk - k
    cb = jnp.repeat(jnp.arange(B, dtype=jnp.int32), k,
                    total_repeat_length=KMAX)
    g = jnp.arange(KMAX, dtype=jnp.int32)
    cr = (g - cume[cb]) * BS_TC
    fl = jnp.zeros((KMAX,), jnp.int32).at[
        jnp.where(k > 0, cumk - 1, KMAX + 1)].set(1, mode="drop")
    meta = jnp.reshape(K, (1,)).astype(jnp.int32)

    mesh = plsc.VectorSubcoreMesh(core_axis_name="c", subcore_axis_name="s")
    sc = pl.kernel(
        _sc_body,
        out_type=jax.ShapeDtypeStruct((NW, B, D), jnp.float32),
        mesh=mesh,
        scratch_types=(
            [pltpu.VMEM((32,), jnp.int32)] * 2
            + [pltpu.VMEM((CH, D), jnp.float32)] * SC_NBUF
            + [pltpu.VMEM((B, D), jnp.float32)]
            + [pltpu.SemaphoreType.DMA] * SC_NBUF
        ),
    )
    partials = sc(x, N, m)

    tcsum = pl.pallas_call(
        _tc_dense_body,
        grid_spec=pltpu.PrefetchScalarGridSpec(
            num_scalar_prefetch=4,
            grid=(1,),
            in_specs=[
                pl.BlockSpec(memory_space=pltpu.MemorySpace.HBM),
            ],
            out_specs=pl.BlockSpec(
                (B, 1, D), lambda _, *refs: (0, 0, 0)),
            scratch_shapes=(
                [pltpu.VMEM((BS_TC, D), jnp.float32)] * TC_NBUF
                + [pltpu.SemaphoreType.DMA] * TC_NBUF
            ),
        ),
        out_shape=jax.ShapeDtypeStruct((B, 1, D), jnp.float32),
    )(cb, cr, fl, meta, x).reshape(B, D)

    nf = N.astype(jnp.float32).reshape(B, 1)
    return pl.pallas_call(
        _combine_body,
        out_shape=jax.ShapeDtypeStruct((B, D), jnp.float32),
    )(partials, tcsum, nf)


# speedup vs baseline: 1.4569x; 1.0160x over previous
import jax
import jax.numpy as jnp
from jax import lax
from jax.experimental import pallas as pl
from jax.experimental.pallas import tpu as pltpu
from jax.experimental.pallas import tpu_sc as plsc

B, S, D = 16, 2048, 512
CH = 64
SC_NBUF = 2
NV = D // 16
HNV = NV // 2
NW = 32
BS_TC = 256
TC_NBUF = 6
KMAX = B * S // BS_TC
FRAC_P, FRAC_Q = 5, 6


def _scalar_at(vec_ref, i):
    return vec_ref[pl.ds(i, 16)][0]


def _sc_body(x_hbm, n_hbm, m_hbm, part_hbm, nvec_ref, mvec_ref,
             b0, b1, part_ref, s0, s1):
    c = lax.axis_index("c")
    s = lax.axis_index("s")
    w = s * 2 + c

    pltpu.sync_copy(n_hbm, nvec_ref.at[pl.ds(0, 16)])
    pltpu.sync_copy(m_hbm, mvec_ref.at[pl.ds(0, 16)])

    def tot_body(j, t):
        return t + (_scalar_at(nvec_ref, j) - _scalar_at(mvec_ref, j))
    T = lax.fori_loop(0, B, tot_body, jnp.int32(0))

    lo = w * T // NW
    hi = (w + 1) * T // NW

    zero = jnp.zeros((16,), jnp.float32)

    def zero_body(r, _):
        for j in range(NV):
            part_ref[r, pl.ds(j * 16, 16)] = zero
        return 0
    lax.fori_loop(0, B, zero_body, 0)

    bufs = (b0, b1)
    sems = (s0, s1)

    def batch_body(i, C):
        n_i = _scalar_at(nvec_ref, i)
        m_i = _scalar_at(mvec_ref, i)
        len_i = n_i - m_i
        a = jnp.maximum(lo, C)
        b = jnp.minimum(hi, C + len_i)

        @pl.when(b > a)
        def _():
            r0 = m_i + (a - C)
            r1 = m_i + (b - C)
            c0 = r0 // CH
            c1 = (r1 + CH - 1) // CH

            def start(chunk, bi):
                @pl.when(chunk < c1)
                def _():
                    pltpu.async_copy(
                        x_hbm.at[i, pl.ds(chunk * CH, CH)], bufs[bi],
                        sems[bi])

            def wait(bi):
                pltpu.make_async_copy(
                    x_hbm.at[i, pl.ds(0, CH)], bufs[bi], sems[bi]).wait()

            for bi in range(SC_NBUF):
                start(c0 + bi, bi)

            def accum_chunk(chunk, bi, accs):
                lo_r = jnp.maximum(r0 - chunk * CH, 0)
                hi_r = jnp.minimum(r1 - chunk * CH, CH)
                buf = bufs[bi]

                @pl.when(chunk < c1)
                def _():
                    wait(bi)

                accs_lo, accs_hi = accs[:HNV], accs[HNV:]

                def row_lo(r, a):
                    return tuple(a[j] + buf[r, pl.ds(j * 16, 16)]
                                 for j in range(HNV))

                def row_hi(r, a):
                    return tuple(a[j] + buf[r, pl.ds((HNV + j) * 16, 16)]
                                 for j in range(HNV))

                accs_lo = lax.fori_loop(lo_r, hi_r, row_lo, accs_lo)
                accs_hi = lax.fori_loop(lo_r, hi_r, row_hi, accs_hi)
                start(chunk + SC_NBUF, bi)
                return accs_lo + accs_hi

            def ring_body(it, accs):
                chunk = c0 + SC_NBUF * it
                for bi in range(SC_NBUF):
                    accs = accum_chunk(chunk + bi, bi, accs)
                return accs

            accs0 = tuple(jnp.zeros((16,), jnp.float32) for _ in range(NV))
            nrings = (c1 - c0 + SC_NBUF - 1) // SC_NBUF
            accs = lax.fori_loop(0, nrings, ring_body, accs0)
            for j in range(NV):
                part_ref[i, pl.ds(j * 16, 16)] = accs[j]

        return C + len_i

    lax.fori_loop(0, B, batch_body, jnp.int32(0))

    pltpu.sync_copy(part_ref, part_hbm.at[w])


def _tc_dense_body(cb_ref, cr_ref, fl_ref, meta_ref, x_hbm, o_ref, *rest):
    bufs = rest[:TC_NBUF]
    sems = rest[TC_NBUF:]
    K = meta_ref[0]

    def start(g, bi):
        @pl.when(g < K)
        def _():
            row = pl.multiple_of(cr_ref[g], BS_TC)
            pltpu.make_async_copy(
                x_hbm.at[cb_ref[g], pl.ds(row, BS_TC)],
                bufs[bi], sems[bi]).start()

    def wait(bi):
        pltpu.make_async_copy(
            x_hbm.at[0, pl.ds(0, BS_TC)], bufs[bi], sems[bi]).wait()

    for bi in range(TC_NBUF):
        start(bi, bi)
    o_ref[...] = jnp.zeros_like(o_ref)

    def step(g, bi, acc):
        valid = g < K

        @pl.when(valid)
        def _():
            wait(bi)

        buf = bufs[bi]
        ps = [buf[pl.ds(p * 8, 8), :] for p in range(4)]
        for r in range(32, BS_TC, 32):
            for p in range(4):
                ps[p] = ps[p] + buf[pl.ds(r + p * 8, 8), :]
        csum = (ps[0] + ps[1]) + (ps[2] + ps[3])
        tot = acc + jnp.where(valid, csum, jnp.zeros_like(csum))
        start(g + TC_NBUF, bi)
        flush = jnp.logical_and(fl_ref[g] == 1, valid)

        @pl.when(flush)
        def _():
            o_ref[pl.ds(cb_ref[g], 1)] = jnp.sum(
                tot, axis=0, keepdims=True).reshape(1, 1, D)

        return jnp.where(flush, jnp.zeros_like(tot), tot)

    def ring_body(it, acc):
        g = TC_NBUF * it
        for bi in range(TC_NBUF):
            acc = step(g + bi, bi, acc)
        return acc

    nrings = (K + TC_NBUF - 1) // TC_NBUF
    lax.fori_loop(0, nrings, ring_body, jnp.zeros((8, D), jnp.float32))


def _combine_body(part_ref, tc_ref, nf_ref, out_ref):
    out_ref[...] = (jnp.sum(part_ref[...], axis=0) + tc_ref[...]) / nf_ref[...]


def kernel(x, N):
    k = (FRAC_P * N) // (FRAC_Q * BS_TC)
    m = k * BS_TC

    K = jnp.sum(k)
    cumk = jnp.cumsum(k)
    cume = cumk - k
    cb = jnp.repeat(jnp.arange(B, dtype=jnp.int32), k,
                    total_repeat_length=KMAX)
    g = jnp.arange(KMAX, dtype=jnp.int32)
    cr = (g - cume[cb]) * BS_TC
    fl = jnp.zeros((KMAX,), jnp.int32).at[
        jnp.where(k > 0, cumk - 1, KMAX + 1)].set(1, mode="drop")
    meta = jnp.reshape(K, (1,)).astype(jnp.int32)

    mesh = plsc.VectorSubcoreMesh(core_axis_name="c", subcore_axis_name="s")
    sc = pl.kernel(
        _sc_body,
        out_type=jax.ShapeDtypeStruct((NW, B, D), jnp.float32),
        mesh=mesh,
        scratch_types=(
            [pltpu.VMEM((32,), jnp.int32)] * 2
            + [pltpu.VMEM((CH, D), jnp.float32)] * SC_NBUF
            + [pltpu.VMEM((B, D), jnp.float32)]
            + [pltpu.SemaphoreType.DMA] * SC_NBUF
        ),
    )
    partials = sc(x, N, m)

    tcsum = pl.pallas_call(
        _tc_dense_body,
        grid_spec=pltpu.PrefetchScalarGridSpec(
            num_scalar_prefetch=4,
            grid=(1,),
            in_specs=[
                pl.BlockSpec(memory_space=pltpu.MemorySpace.HBM),
            ],
            out_specs=pl.BlockSpec(
                (B, 1, D), lambda _, *refs: (0, 0, 0)),
            scratch_shapes=(
                [pltpu.VMEM((BS_TC, D), jnp.float32)] * TC_NBUF
                + [pltpu.SemaphoreType.DMA] * TC_NBUF
            ),
        ),
        out_shape=jax.ShapeDtypeStruct((B, 1, D), jnp.float32),
    )(cb, cr, fl, meta, x).reshape(B, D)

    nf = N.astype(jnp.float32).reshape(B, 1)
    return pl.pallas_call(
        _combine_body,
        out_shape=jax.ShapeDtypeStruct((B, D), jnp.float32),
    )(partials, tcsum, nf)


# speedup vs baseline: 2.5878x; 1.7762x over previous
import jax
import jax.numpy as jnp
from jax import lax
from jax.experimental import pallas as pl
from jax.experimental.pallas import tpu as pltpu
from jax.experimental.pallas import tpu_sc as plsc

B, S, D = 16, 2048, 512
CH = 64
SC_NBUF = 2
NV = D // 16
HNV = NV // 2
NW = 32
BS_TC = 256
TC_NBUF = 6
FRAC_P, FRAC_Q = 5, 6


def _scalar_at(vec_ref, i):
    return vec_ref[pl.ds(i, 16)][0]


def _sc_body(x_hbm, n_hbm, m_hbm, part_hbm, nvec_ref, mvec_ref,
             b0, b1, part_ref, s0, s1):
    c = lax.axis_index("c")
    s = lax.axis_index("s")
    w = s * 2 + c

    pltpu.sync_copy(n_hbm, nvec_ref.at[pl.ds(0, 16)])
    pltpu.sync_copy(m_hbm, mvec_ref.at[pl.ds(0, 16)])

    def tot_body(j, t):
        return t + (_scalar_at(nvec_ref, j) - _scalar_at(mvec_ref, j))
    T = lax.fori_loop(0, B, tot_body, jnp.int32(0))

    lo = w * T // NW
    hi = (w + 1) * T // NW

    zero = jnp.zeros((16,), jnp.float32)

    def zero_body(r, _):
        for j in range(NV):
            part_ref[r, pl.ds(j * 16, 16)] = zero
        return 0
    lax.fori_loop(0, B, zero_body, 0)

    bufs = (b0, b1)
    sems = (s0, s1)

    def batch_body(i, C):
        n_i = _scalar_at(nvec_ref, i)
        m_i = _scalar_at(mvec_ref, i)
        len_i = n_i - m_i
        a = jnp.maximum(lo, C)
        b = jnp.minimum(hi, C + len_i)

        @pl.when(b > a)
        def _():
            r0 = m_i + (a - C)
            r1 = m_i + (b - C)
            c0 = r0 // CH
            c1 = (r1 + CH - 1) // CH

            def start(chunk, bi):
                @pl.when(chunk < c1)
                def _():
                    pltpu.async_copy(
                        x_hbm.at[i, pl.ds(chunk * CH, CH)], bufs[bi],
                        sems[bi])

            def wait(bi):
                pltpu.make_async_copy(
                    x_hbm.at[i, pl.ds(0, CH)], bufs[bi], sems[bi]).wait()

            for bi in range(SC_NBUF):
                start(c0 + bi, bi)

            def accum_chunk(chunk, bi, accs):
                lo_r = jnp.maximum(r0 - chunk * CH, 0)
                hi_r = jnp.minimum(r1 - chunk * CH, CH)
                buf = bufs[bi]

                @pl.when(chunk < c1)
                def _():
                    wait(bi)

                accs_lo, accs_hi = accs[:HNV], accs[HNV:]

                def row_lo(r, a):
                    return tuple(a[j] + buf[r, pl.ds(j * 16, 16)]
                                 for j in range(HNV))

                def row_hi(r, a):
                    return tuple(a[j] + buf[r, pl.ds((HNV + j) * 16, 16)]
                                 for j in range(HNV))

                accs_lo = lax.fori_loop(lo_r, hi_r, row_lo, accs_lo)
                accs_hi = lax.fori_loop(lo_r, hi_r, row_hi, accs_hi)
                start(chunk + SC_NBUF, bi)
                return accs_lo + accs_hi

            def ring_body(it, accs):
                chunk = c0 + SC_NBUF * it
                for bi in range(SC_NBUF):
                    accs = accum_chunk(chunk + bi, bi, accs)
                return accs

            accs0 = tuple(jnp.zeros((16,), jnp.float32) for _ in range(NV))
            nrings = (c1 - c0 + SC_NBUF - 1) // SC_NBUF
            accs = lax.fori_loop(0, nrings, ring_body, accs0)
            for j in range(NV):
                part_ref[i, pl.ds(j * 16, 16)] = accs[j]

        return C + len_i

    lax.fori_loop(0, B, batch_body, jnp.int32(0))

    pltpu.sync_copy(part_ref, part_hbm.at[w])


def _tc_dense_body(k_ref, ck_ref, x_hbm, o_ref, *rest):
    bufs = rest[:TC_NBUF]
    sems = rest[TC_NBUF:]
    K = ck_ref[B - 1]

    def locate(g):
        i = jnp.int32(0)
        for j in range(B):
            i = i + (g >= ck_ref[j]).astype(jnp.int32)
        ki = k_ref[i]
        c = g - (ck_ref[i] - ki)
        return i, c, c == ki - 1

    def start(g, bi):
        @pl.when(g < K)
        def _():
            i, c, _ = locate(g)
            row = pl.multiple_of(c * BS_TC, BS_TC)
            pltpu.make_async_copy(
                x_hbm.at[i, pl.ds(row, BS_TC)],
                bufs[bi], sems[bi]).start()

    def wait(bi):
        pltpu.make_async_copy(
            x_hbm.at[0, pl.ds(0, BS_TC)], bufs[bi], sems[bi]).wait()

    for bi in range(TC_NBUF):
        start(bi, bi)
    o_ref[...] = jnp.zeros_like(o_ref)

    def step(g, bi, acc):
        valid = g < K

        @pl.when(valid)
        def _():
            wait(bi)

        buf = bufs[bi]
        ps = [buf[pl.ds(p * 8, 8), :] for p in range(4)]
        for r in range(32, BS_TC, 32):
            for p in range(4):
                ps[p] = ps[p] + buf[pl.ds(r + p * 8, 8), :]
        csum = (ps[0] + ps[1]) + (ps[2] + ps[3])
        tot = acc + jnp.where(valid, csum, jnp.zeros_like(csum))
        start(g + TC_NBUF, bi)
        ig, _, last = locate(g)
        flush = jnp.logical_and(last, valid)

        @pl.when(flush)
        def _():
            o_ref[pl.ds(ig, 1)] = jnp.sum(
                tot, axis=0, keepdims=True).reshape(1, 1, D)

        return jnp.where(flush, jnp.zeros_like(tot), tot)

    def ring_body(it, acc):
        g = TC_NBUF * it
        for bi in range(TC_NBUF):
            acc = step(g + bi, bi, acc)
        return acc

    nrings = (K + TC_NBUF - 1) // TC_NBUF
    lax.fori_loop(0, nrings, ring_body, jnp.zeros((8, D), jnp.float32))


def _combine_body(part_ref, tc_ref, nf_ref, out_ref):
    out_ref[...] = (jnp.sum(part_ref[...], axis=0)
                    + tc_ref[:, 0, :]) / nf_ref[...]


def kernel(x, N):
    k = (FRAC_P * N) // (FRAC_Q * BS_TC)
    m = k * BS_TC
    cumk = jnp.cumsum(k)

    mesh = plsc.VectorSubcoreMesh(core_axis_name="c", subcore_axis_name="s")
    sc = pl.kernel(
        _sc_body,
        out_type=jax.ShapeDtypeStruct((NW, B, D), jnp.float32),
        mesh=mesh,
        scratch_types=(
            [pltpu.VMEM((32,), jnp.int32)] * 2
            + [pltpu.VMEM((CH, D), jnp.float32)] * SC_NBUF
            + [pltpu.VMEM((B, D), jnp.float32)]
            + [pltpu.SemaphoreType.DMA] * SC_NBUF
        ),
    )
    partials = sc(x, N, m)

    tcsum = pl.pallas_call(
        _tc_dense_body,
        grid_spec=pltpu.PrefetchScalarGridSpec(
            num_scalar_prefetch=2,
            grid=(1,),
            in_specs=[
                pl.BlockSpec(memory_space=pltpu.MemorySpace.HBM),
            ],
            out_specs=pl.BlockSpec(
                (B, 1, D), lambda _, *refs: (0, 0, 0)),
            scratch_shapes=(
                [pltpu.VMEM((BS_TC, D), jnp.float32)] * TC_NBUF
                + [pltpu.SemaphoreType.DMA] * TC_NBUF
            ),
        ),
        out_shape=jax.ShapeDtypeStruct((B, 1, D), jnp.float32),
    )(k, cumk, x)

    nf = N.astype(jnp.float32).reshape(B, 1)
    return pl.pallas_call(
        _combine_body,
        out_shape=jax.ShapeDtypeStruct((B, D), jnp.float32),
    )(partials, tcsum, nf)


# speedup vs baseline: 2.6272x; 1.0152x over previous
import jax
import jax.numpy as jnp
from jax import lax
from jax.experimental import pallas as pl
from jax.experimental.pallas import tpu as pltpu
from jax.experimental.pallas import tpu_sc as plsc

B, S, D = 16, 2048, 512
CH = 64
SC_NBUF = 2
NV = D // 16
HNV = NV // 2
NW = 32
BS_TC = 256
TC_NBUF = 6
FRAC_P, FRAC_Q = 8, 9


def _scalar_at(vec_ref, i):
    return vec_ref[pl.ds(i, 16)][0]


def _sc_body(x_hbm, n_hbm, m_hbm, part_hbm, nvec_ref, mvec_ref,
             b0, b1, part_ref, s0, s1):
    c = lax.axis_index("c")
    s = lax.axis_index("s")
    w = s * 2 + c

    pltpu.sync_copy(n_hbm, nvec_ref.at[pl.ds(0, 16)])
    pltpu.sync_copy(m_hbm, mvec_ref.at[pl.ds(0, 16)])

    def tot_body(j, t):
        return t + (_scalar_at(nvec_ref, j) - _scalar_at(mvec_ref, j))
    T = lax.fori_loop(0, B, tot_body, jnp.int32(0))

    lo = w * T // NW
    hi = (w + 1) * T // NW

    zero = jnp.zeros((16,), jnp.float32)

    def zero_body(r, _):
        for j in range(NV):
            part_ref[r, pl.ds(j * 16, 16)] = zero
        return 0
    lax.fori_loop(0, B, zero_body, 0)

    bufs = (b0, b1)
    sems = (s0, s1)

    def batch_body(i, C):
        n_i = _scalar_at(nvec_ref, i)
        m_i = _scalar_at(mvec_ref, i)
        len_i = n_i - m_i
        a = jnp.maximum(lo, C)
        b = jnp.minimum(hi, C + len_i)

        @pl.when(b > a)
        def _():
            r0 = m_i + (a - C)
            r1 = m_i + (b - C)
            c0 = r0 // CH
            c1 = (r1 + CH - 1) // CH

            def start(chunk, bi):
                @pl.when(chunk < c1)
                def _():
                    pltpu.async_copy(
                        x_hbm.at[i, pl.ds(chunk * CH, CH)], bufs[bi],
                        sems[bi])

            def wait(bi):
                pltpu.make_async_copy(
                    x_hbm.at[i, pl.ds(0, CH)], bufs[bi], sems[bi]).wait()

            for bi in range(SC_NBUF):
                start(c0 + bi, bi)

            def accum_chunk(chunk, bi, accs):
                lo_r = jnp.maximum(r0 - chunk * CH, 0)
                hi_r = jnp.minimum(r1 - chunk * CH, CH)
                buf = bufs[bi]

                @pl.when(chunk < c1)
                def _():
                    wait(bi)

                accs_lo, accs_hi = accs[:HNV], accs[HNV:]

                def row_lo(r, a):
                    return tuple(a[j] + buf[r, pl.ds(j * 16, 16)]
                                 for j in range(HNV))

                def row_hi(r, a):
                    return tuple(a[j] + buf[r, pl.ds((HNV + j) * 16, 16)]
                                 for j in range(HNV))

                accs_lo = lax.fori_loop(lo_r, hi_r, row_lo, accs_lo)
                accs_hi = lax.fori_loop(lo_r, hi_r, row_hi, accs_hi)
                start(chunk + SC_NBUF, bi)
                return accs_lo + accs_hi

            def ring_body(it, accs):
                chunk = c0 + SC_NBUF * it
                for bi in range(SC_NBUF):
                    accs = accum_chunk(chunk + bi, bi, accs)
                return accs

            accs0 = tuple(jnp.zeros((16,), jnp.float32) for _ in range(NV))
            nrings = (c1 - c0 + SC_NBUF - 1) // SC_NBUF
            accs = lax.fori_loop(0, nrings, ring_body, accs0)
            for j in range(NV):
                part_ref[i, pl.ds(j * 16, 16)] = accs[j]

        return C + len_i

    lax.fori_loop(0, B, batch_body, jnp.int32(0))

    pltpu.sync_copy(part_ref, part_hbm.at[w])


def _tc_dense_body(k_ref, x_hbm, o_ref, *rest):
    bufs = rest[:TC_NBUF]
    semsa = rest[TC_NBUF:2 * TC_NBUF]
    semsb = rest[2 * TC_NBUF:]
    HB = BS_TC // 2

    def locate(g):
        i = jnp.int32(0)
        run = jnp.int32(0)
        pre = jnp.int32(0)
        for j in range(B):
            run = run + k_ref[j]
            hit = (g >= run).astype(jnp.int32)
            i = i + hit
            pre = pre + hit * k_ref[j]
        ki = k_ref[jnp.minimum(i, B - 1)]
        c = g - pre
        return jnp.minimum(i, B - 1), c, c == ki - 1

    def total():
        t = jnp.int32(0)
        for j in range(B):
            t = t + k_ref[j]
        return t

    K = total()

    def start(g, bi):
        @pl.when(g < K)
        def _():
            i, c, _ = locate(g)
            row = pl.multiple_of(c * BS_TC, BS_TC)
            pltpu.make_async_copy(
                x_hbm.at[i, pl.ds(row, HB)],
                bufs[bi].at[pl.ds(0, HB)], semsa[bi]).start()
            row2 = pl.multiple_of(row + HB, HB)
            pltpu.make_async_copy(
                x_hbm.at[i, pl.ds(row2, HB)],
                bufs[bi].at[pl.ds(HB, HB)], semsb[bi]).start()

    def wait(bi):
        pltpu.make_async_copy(
            x_hbm.at[0, pl.ds(0, HB)],
            bufs[bi].at[pl.ds(0, HB)], semsa[bi]).wait()
        pltpu.make_async_copy(
            x_hbm.at[0, pl.ds(0, HB)],
            bufs[bi].at[pl.ds(HB, HB)], semsb[bi]).wait()

    for bi in range(TC_NBUF):
        start(bi, bi)
    o_ref[...] = jnp.zeros_like(o_ref)

    def step(g, bi, acc):
        valid = g < K

        @pl.when(valid)
        def _():
            wait(bi)

        buf = bufs[bi]
        ps = [buf[pl.ds(p * 8, 8), :] for p in range(4)]
        for r in range(32, BS_TC, 32):
            for p in range(4):
                ps[p] = ps[p] + buf[pl.ds(r + p * 8, 8), :]
        csum = (ps[0] + ps[1]) + (ps[2] + ps[3])
        tot = acc + jnp.where(valid, csum, jnp.zeros_like(csum))
        start(g + TC_NBUF, bi)
        ig, _, last = locate(g)
        flush = jnp.logical_and(last, valid)

        @pl.when(flush)
        def _():
            o_ref[pl.ds(ig, 1)] = jnp.sum(
                tot, axis=0, keepdims=True).reshape(1, 1, D)

        return jnp.where(flush, jnp.zeros_like(tot), tot)

    def ring_body(it, acc):
        g = TC_NBUF * it
        for bi in range(TC_NBUF):
            acc = step(g + bi, bi, acc)
        return acc

    nrings = (K + TC_NBUF - 1) // TC_NBUF
    lax.fori_loop(0, nrings, ring_body, jnp.zeros((8, D), jnp.float32))


def _combine_body(part_ref, tc_ref, nf_ref, out_ref):
    out_ref[...] = (jnp.sum(part_ref[...], axis=0)
                    + tc_ref[:, 0, :]) / nf_ref[...]


def kernel(x, N):
    k = (FRAC_P * N) // (FRAC_Q * BS_TC)
    m = k * BS_TC

    mesh = plsc.VectorSubcoreMesh(core_axis_name="c", subcore_axis_name="s")
    sc = pl.kernel(
        _sc_body,
        out_type=jax.ShapeDtypeStruct((NW, B, D), jnp.float32),
        mesh=mesh,
        scratch_types=(
            [pltpu.VMEM((32,), jnp.int32)] * 2
            + [pltpu.VMEM((CH, D), jnp.float32)] * SC_NBUF
            + [pltpu.VMEM((B, D), jnp.float32)]
            + [pltpu.SemaphoreType.DMA] * SC_NBUF
        ),
    )
    partials = sc(x, N, m)

    tcsum = pl.pallas_call(
        _tc_dense_body,
        grid_spec=pltpu.PrefetchScalarGridSpec(
            num_scalar_prefetch=1,
            grid=(1,),
            in_specs=[
                pl.BlockSpec(memory_space=pltpu.MemorySpace.HBM),
            ],
            out_specs=pl.BlockSpec(
                (B, 1, D), lambda _, *refs: (0, 0, 0)),
            scratch_shapes=(
                [pltpu.VMEM((BS_TC, D), jnp.float32)] * TC_NBUF
                + [pltpu.SemaphoreType.DMA] * (2 * TC_NBUF)
            ),
        ),
        out_shape=jax.ShapeDtypeStruct((B, 1, D), jnp.float32),
    )(k, x)

    nf = N.astype(jnp.float32).reshape(B, 1)
    return pl.pallas_call(
        _combine_body,
        out_shape=jax.ShapeDtypeStruct((B, D), jnp.float32),
    )(partials, tcsum, nf)


# speedup vs baseline: 2.6806x; 1.0203x over previous
import jax
import jax.numpy as jnp
from jax import lax
from jax.experimental import pallas as pl
from jax.experimental.pallas import tpu as pltpu
from jax.experimental.pallas import tpu_sc as plsc

B, S, D = 16, 2048, 512
CH = 64
SC_NBUF = 2
NV = D // 16
HNV = NV // 2
NW = 32
BS_TC = 256
TC_NBUF = 6
FRAC_P, FRAC_Q = 8, 9


def _scalar_at(vec_ref, i):
    return vec_ref[pl.ds(i, 16)][0]


def _sc_body(x_hbm, n_hbm, m_hbm, part_hbm, nvec_ref, mvec_ref,
             b0, b1, part_ref, s0, s1):
    c = lax.axis_index("c")
    s = lax.axis_index("s")
    w = s * 2 + c

    pltpu.sync_copy(n_hbm, nvec_ref.at[pl.ds(0, 16)])
    pltpu.sync_copy(m_hbm, mvec_ref.at[pl.ds(0, 16)])

    def tot_body(j, t):
        return t + (_scalar_at(nvec_ref, j) - _scalar_at(mvec_ref, j))
    T = lax.fori_loop(0, B, tot_body, jnp.int32(0))

    lo = w * T // NW
    hi = (w + 1) * T // NW

    zero = jnp.zeros((16,), jnp.float32)

    def zero_body(r, _):
        for j in range(NV):
            part_ref[r, pl.ds(j * 16, 16)] = zero
        return 0
    lax.fori_loop(0, B, zero_body, 0)

    bufs = (b0, b1)
    sems = (s0, s1)

    def batch_body(i, C):
        n_i = _scalar_at(nvec_ref, i)
        m_i = _scalar_at(mvec_ref, i)
        len_i = n_i - m_i
        a = jnp.maximum(lo, C)
        b = jnp.minimum(hi, C + len_i)

        @pl.when(b > a)
        def _():
            r0 = m_i + (a - C)
            r1 = m_i + (b - C)
            c0 = r0 // CH
            c1 = (r1 + CH - 1) // CH

            def start(chunk, bi):
                @pl.when(chunk < c1)
                def _():
                    pltpu.async_copy(
                        x_hbm.at[i, pl.ds(chunk * CH, CH)], bufs[bi],
                        sems[bi])

            def wait(bi):
                pltpu.make_async_copy(
                    x_hbm.at[i, pl.ds(0, CH)], bufs[bi], sems[bi]).wait()

            for bi in range(SC_NBUF):
                start(c0 + bi, bi)

            def accum_chunk(chunk, bi, accs):
                lo_r = jnp.maximum(r0 - chunk * CH, 0)
                hi_r = jnp.minimum(r1 - chunk * CH, CH)
                buf = bufs[bi]

                @pl.when(chunk < c1)
                def _():
                    wait(bi)

                accs_lo, accs_hi = accs[:HNV], accs[HNV:]

                def row_lo(r, a):
                    return tuple(a[j] + buf[r, pl.ds(j * 16, 16)]
                                 for j in range(HNV))

                def row_hi(r, a):
                    return tuple(a[j] + buf[r, pl.ds((HNV + j) * 16, 16)]
                                 for j in range(HNV))

                accs_lo = lax.fori_loop(lo_r, hi_r, row_lo, accs_lo)
                accs_hi = lax.fori_loop(lo_r, hi_r, row_hi, accs_hi)
                start(chunk + SC_NBUF, bi)
                return accs_lo + accs_hi

            def ring_body(it, accs):
                chunk = c0 + SC_NBUF * it
                for bi in range(SC_NBUF):
                    accs = accum_chunk(chunk + bi, bi, accs)
                return accs

            accs0 = tuple(jnp.zeros((16,), jnp.float32) for _ in range(NV))
            nrings = (c1 - c0 + SC_NBUF - 1) // SC_NBUF
            accs = lax.fori_loop(0, nrings, ring_body, accs0)
            for j in range(NV):
                part_ref[i, pl.ds(j * 16, 16)] = accs[j]

        return C + len_i

    lax.fori_loop(0, B, batch_body, jnp.int32(0))

    pltpu.sync_copy(part_ref, part_hbm.at[w])


def _tc_dense_body(k_ref, x_hbm, o_ref, *rest):
    bufs = rest[:TC_NBUF]
    semsa = rest[TC_NBUF:2 * TC_NBUF]
    semsb = rest[2 * TC_NBUF:]
    HB = BS_TC // 2

    def locate(g):
        i = jnp.int32(0)
        run = jnp.int32(0)
        pre = jnp.int32(0)
        for j in range(B):
            run = run + k_ref[j]
            hit = (g >= run).astype(jnp.int32)
            i = i + hit
            pre = pre + hit * k_ref[j]
        ki = k_ref[jnp.minimum(i, B - 1)]
        c = g - pre
        return jnp.minimum(i, B - 1), c, c == ki - 1

    def total():
        t = jnp.int32(0)
        for j in range(B):
            t = t + k_ref[j]
        return t

    K = total()

    def start(g, bi):
        @pl.when(g < K)
        def _():
            i, c, _ = locate(g)
            row = pl.multiple_of(c * BS_TC, BS_TC)
            pltpu.make_async_copy(
                x_hbm.at[i, pl.ds(row, BS_TC)],
                bufs[bi], semsa[bi]).start()

    def wait(bi):
        pltpu.make_async_copy(
            x_hbm.at[0, pl.ds(0, BS_TC)],
            bufs[bi], semsa[bi]).wait()

    for bi in range(TC_NBUF):
        start(bi, bi)
    o_ref[...] = jnp.zeros_like(o_ref)

    def step(g, bi, acc):
        valid = g < K

        @pl.when(valid)
        def _():
            wait(bi)

        buf = bufs[bi]
        ps = [buf[pl.ds(p * 8, 8), :] for p in range(4)]
        for r in range(32, BS_TC, 32):
            for p in range(4):
                ps[p] = ps[p] + buf[pl.ds(r + p * 8, 8), :]
        csum = (ps[0] + ps[1]) + (ps[2] + ps[3])
        tot = acc + jnp.where(valid, csum, jnp.zeros_like(csum))
        start(g + TC_NBUF, bi)
        ig, _, last = locate(g)
        flush = jnp.logical_and(last, valid)

        @pl.when(flush)
        def _():
            o_ref[pl.ds(ig, 1)] = jnp.sum(
                tot, axis=0, keepdims=True).reshape(1, 1, D)

        return jnp.where(flush, jnp.zeros_like(tot), tot)

    def ring_body(it, acc):
        g = TC_NBUF * it
        for bi in range(TC_NBUF):
            acc = step(g + bi, bi, acc)
        return acc

    nrings = (K + TC_NBUF - 1) // TC_NBUF
    lax.fori_loop(0, nrings, ring_body, jnp.zeros((8, D), jnp.float32))


def _combine_body(part_ref, tc_ref, nf_ref, out_ref):
    out_ref[...] = (jnp.sum(part_ref[...], axis=0)
                    + tc_ref[:, 0, :]) / nf_ref[...]


def kernel(x, N):
    k = (FRAC_P * N) // (FRAC_Q * BS_TC)
    m = k * BS_TC

    mesh = plsc.VectorSubcoreMesh(core_axis_name="c", subcore_axis_name="s")
    sc = pl.kernel(
        _sc_body,
        out_type=jax.ShapeDtypeStruct((NW, B, D), jnp.float32),
        mesh=mesh,
        scratch_types=(
            [pltpu.VMEM((32,), jnp.int32)] * 2
            + [pltpu.VMEM((CH, D), jnp.float32)] * SC_NBUF
            + [pltpu.VMEM((B, D), jnp.float32)]
            + [pltpu.SemaphoreType.DMA] * SC_NBUF
        ),
    )
    tcsum_call = pl.pallas_call(
        _tc_dense_body,
        grid_spec=pltpu.PrefetchScalarGridSpec(
            num_scalar_prefetch=1,
            grid=(1,),
            in_specs=[
                pl.BlockSpec(memory_space=pltpu.MemorySpace.HBM),
            ],
            out_specs=pl.BlockSpec(
                (B, 1, D), lambda _, *refs: (0, 0, 0)),
            scratch_shapes=(
                [pltpu.VMEM((BS_TC, D), jnp.float32)] * TC_NBUF
                + [pltpu.SemaphoreType.DMA] * (2 * TC_NBUF)
            ),
        ),
        out_shape=jax.ShapeDtypeStruct((B, 1, D), jnp.float32),
    )
    tcsum = tcsum_call(k, x)
    partials = sc(x, N, m)

    nf = N.astype(jnp.float32).reshape(B, 1)
    return pl.pallas_call(
        _combine_body,
        out_shape=jax.ShapeDtypeStruct((B, D), jnp.float32),
    )(partials, tcsum, nf)
